# baseline (speedup 1.0000x reference)
import functools

import jax
import jax.numpy as jnp
from jax import lax
from jax.experimental import pallas as pl
from jax.experimental.pallas import tpu as pltpu
from jax.experimental.pallas import tpu_sc as plsc

_SCALAR = 30.0
_C_BLOCK = 3072


def _make_target_row_gather(n, nf):
    info = plsc.get_sparse_core_info()
    nc, ns = info.num_cores, info.num_subcores
    nw = nc * ns
    assert n % nw == 0
    bpw = n // nw
    mesh = plsc.VectorSubcoreMesh(core_axis_name="c", subcore_axis_name="s")

    @functools.partial(
        pl.kernel,
        mesh=mesh,
        out_type=jax.ShapeDtypeStruct((n, nf), jnp.float32),
        scratch_types=[
            pltpu.VMEM((bpw,), jnp.int32),
            pltpu.VMEM((bpw, nf), jnp.float32),
            pltpu.SemaphoreType.DMA,
        ],
    )
    def gather_kernel(t_hbm, lut_hbm, out_hbm, idx_v, rows_v, sem):
        wid = lax.axis_index("s") * nc + lax.axis_index("c")
        base = wid * bpw
        pltpu.sync_copy(t_hbm.at[pl.ds(base, bpw)], idx_v)
        pltpu.async_copy(lut_hbm.at[idx_v], rows_v, sem).wait()
        pltpu.sync_copy(rows_v, out_hbm.at[pl.ds(base, bpw)])

    return gather_kernel


def _oim_body(f_ref, g_ref, l_ref, out_ref, loss_ref, m_ref, s_ref,
              *, num_classes, c_block):
    i = pl.program_id(0)
    nblk = pl.num_programs(0)
    n = f_ref.shape[0]

    @pl.when(i == 0)
    def _init():
        m_ref[...] = jnp.full((1, n), -jnp.inf, jnp.float32)
        s_ref[...] = jnp.zeros((1, n), jnp.float32)

    s = jax.lax.dot_general(
        l_ref[...], f_ref[...], (((1,), (1,)), ((), ())),
        preferred_element_type=jnp.float32)
    out_ref[...] = s

    ones = jnp.ones((1, c_block), jnp.float32)

    def _colmax(x):
        c = x.shape[0]
        while c > 8 and (c // 2) % 8 == 0:
            c //= 2
            x = jnp.maximum(x[:c], x[c:])
        return jnp.max(x, axis=0, keepdims=True)

    def _update(sm):
        bmax = _colmax(sm)
        m_old = m_ref[...]
        m_new = jnp.maximum(m_old, bmax)
        m_ref[...] = m_new
        e = jnp.exp(sm - m_new)
        esum = jax.lax.dot_general(
            ones, e, (((1,), (0,)), ((), ())),
            preferred_element_type=jnp.float32)
        s_ref[...] = s_ref[...] * jnp.exp(m_old - m_new) + esum

    @pl.when(i < nblk - 1)
    def _main():
        _update(s)

    @pl.when(i == nblk - 1)
    def _edge():
        row = jax.lax.broadcasted_iota(jnp.int32, (c_block, 1), 0) + i * c_block
        _update(jnp.where(row < num_classes, s, -jnp.inf))
        lse = m_ref[...] + jnp.log(s_ref[...])
        tsum = jnp.sum(f_ref[...] * g_ref[...])
        loss_ref[...] = ((jnp.sum(lse) - tsum) / n).reshape(1, 1)


def kernel(features, targets, lut):
    n, nf = features.shape
    num_classes = lut.shape[0]
    c_block = _C_BLOCK
    nblk = pl.cdiv(num_classes, c_block)

    fs = features * jnp.float32(_SCALAR)
    g_rows = _make_target_row_gather(n, nf)(targets.astype(jnp.int32), lut)

    scaled_t, loss = pl.pallas_call(
        functools.partial(_oim_body, num_classes=num_classes, c_block=c_block),
        grid=(nblk,),
        in_specs=[
            pl.BlockSpec((n, nf), lambda i: (0, 0)),
            pl.BlockSpec((n, nf), lambda i: (0, 0)),
            pl.BlockSpec((c_block, nf), lambda i: (i, 0)),
        ],
        out_specs=[
            pl.BlockSpec((c_block, n), lambda i: (i, 0)),
            pl.BlockSpec((1, 1), lambda i: (0, 0)),
        ],
        out_shape=[
            jax.ShapeDtypeStruct((num_classes, n), jnp.float32),
            jax.ShapeDtypeStruct((1, 1), jnp.float32),
        ],
        scratch_shapes=[
            pltpu.VMEM((1, n), jnp.float32),
            pltpu.VMEM((1, n), jnp.float32),
        ],
    )(fs, g_rows, lut)
    return (loss.reshape(()), scaled_t.T)


# speedup vs baseline: 1.0121x; 1.0121x over previous
import functools

import jax
import jax.numpy as jnp
from jax import lax
from jax.experimental import pallas as pl
from jax.experimental.pallas import tpu as pltpu
from jax.experimental.pallas import tpu_sc as plsc

_SCALAR = 30.0
_C_BLOCK = 3456


def _make_target_row_gather(n, nf):
    info = plsc.get_sparse_core_info()
    nc, ns = info.num_cores, info.num_subcores
    nw = nc * ns
    assert n % nw == 0
    bpw = n // nw
    mesh = plsc.VectorSubcoreMesh(core_axis_name="c", subcore_axis_name="s")

    @functools.partial(
        pl.kernel,
        mesh=mesh,
        out_type=jax.ShapeDtypeStruct((n, nf), jnp.float32),
        scratch_types=[
            pltpu.VMEM((bpw,), jnp.int32),
            pltpu.VMEM((bpw, nf), jnp.float32),
            pltpu.SemaphoreType.DMA,
        ],
    )
    def gather_kernel(t_hbm, lut_hbm, out_hbm, idx_v, rows_v, sem):
        wid = lax.axis_index("s") * nc + lax.axis_index("c")
        base = wid * bpw
        pltpu.sync_copy(t_hbm.at[pl.ds(base, bpw)], idx_v)
        pltpu.async_copy(lut_hbm.at[idx_v], rows_v, sem).wait()
        pltpu.sync_copy(rows_v, out_hbm.at[pl.ds(base, bpw)])

    return gather_kernel


def _oim_body(f_ref, g_ref, l_ref, out_ref, loss_ref, m_ref, s_ref,
              *, num_classes, c_block):
    i = pl.program_id(0)
    nblk = pl.num_programs(0)
    n = f_ref.shape[0]

    @pl.when(i == 0)
    def _init():
        m_ref[...] = jnp.full((1, n), -jnp.inf, jnp.float32)
        s_ref[...] = jnp.zeros((1, n), jnp.float32)

    s = jax.lax.dot_general(
        l_ref[...], f_ref[...], (((1,), (1,)), ((), ())),
        preferred_element_type=jnp.float32)
    out_ref[...] = s

    ones = jnp.ones((1, c_block), jnp.float32)

    def _update(sm):
        bmax = jnp.max(sm, axis=0, keepdims=True)
        m_old = m_ref[...]
        m_new = jnp.maximum(m_old, bmax)
        m_ref[...] = m_new
        e = jnp.exp(sm - m_new)
        esum = jax.lax.dot_general(
            ones, e, (((1,), (0,)), ((), ())),
            preferred_element_type=jnp.float32)
        s_ref[...] = s_ref[...] * jnp.exp(m_old - m_new) + esum

    @pl.when(i < nblk - 1)
    def _main():
        _update(s)

    @pl.when(i == nblk - 1)
    def _edge():
        row = jax.lax.broadcasted_iota(jnp.int32, (c_block, 1), 0) + i * c_block
        _update(jnp.where(row < num_classes, s, -jnp.inf))
        lse = m_ref[...] + jnp.log(s_ref[...])
        tsum = jnp.sum(f_ref[...] * g_ref[...])
        loss_ref[...] = ((jnp.sum(lse) - tsum) / n).reshape(1, 1)


def kernel(features, targets, lut):
    n, nf = features.shape
    num_classes = lut.shape[0]
    c_block = _C_BLOCK
    nblk = pl.cdiv(num_classes, c_block)

    fs = features * jnp.float32(_SCALAR)
    g_rows = _make_target_row_gather(n, nf)(targets.astype(jnp.int32), lut)

    scaled_t, loss = pl.pallas_call(
        functools.partial(_oim_body, num_classes=num_classes, c_block=c_block),
        grid=(nblk,),
        in_specs=[
            pl.BlockSpec((n, nf), lambda i: (0, 0)),
            pl.BlockSpec((n, nf), lambda i: (0, 0)),
            pl.BlockSpec((c_block, nf), lambda i: (i, 0)),
        ],
        out_specs=[
            pl.BlockSpec((c_block, n), lambda i: (i, 0)),
            pl.BlockSpec((1, 1), lambda i: (0, 0)),
        ],
        out_shape=[
            jax.ShapeDtypeStruct((num_classes, n), jnp.float32),
            jax.ShapeDtypeStruct((1, 1), jnp.float32),
        ],
        scratch_shapes=[
            pltpu.VMEM((1, n), jnp.float32),
            pltpu.VMEM((1, n), jnp.float32),
        ],
        compiler_params=pltpu.CompilerParams(
            vmem_limit_bytes=60 * 1024 * 1024),
    )(fs, g_rows, lut)
    return (loss.reshape(()), scaled_t.T)


# speedup vs baseline: 1.0185x; 1.0063x over previous
import functools

import jax
import jax.numpy as jnp
from jax import lax
from jax.experimental import pallas as pl
from jax.experimental.pallas import tpu as pltpu
from jax.experimental.pallas import tpu_sc as plsc

_SCALAR = 30.0
_C_BLOCK = 3456


def _make_target_row_gather(n, nf):
    info = plsc.get_sparse_core_info()
    nc, ns = info.num_cores, info.num_subcores
    nw = nc * ns
    assert n % nw == 0
    bpw = n // nw
    mesh = plsc.VectorSubcoreMesh(core_axis_name="c", subcore_axis_name="s")

    @functools.partial(
        pl.kernel,
        mesh=mesh,
        out_type=jax.ShapeDtypeStruct((n, nf), jnp.float32),
        scratch_types=[
            pltpu.VMEM((bpw,), jnp.int32),
            pltpu.VMEM((bpw, nf), jnp.float32),
            pltpu.SemaphoreType.DMA,
        ],
    )
    def gather_kernel(t_hbm, lut_hbm, out_hbm, idx_v, rows_v, sem):
        wid = lax.axis_index("s") * nc + lax.axis_index("c")
        base = wid * bpw
        pltpu.sync_copy(t_hbm.at[pl.ds(base, bpw)], idx_v)
        pltpu.async_copy(lut_hbm.at[idx_v], rows_v, sem).wait()
        pltpu.sync_copy(rows_v, out_hbm.at[pl.ds(base, bpw)])

    return gather_kernel


def _oim_body(f_ref, l_ref, out_ref, lsesum_ref, m_ref, s_ref,
              *, num_classes, c_block):
    i = pl.program_id(0)
    nblk = pl.num_programs(0)
    n = f_ref.shape[0]

    @pl.when(i == 0)
    def _init():
        m_ref[...] = jnp.full((1, n), -jnp.inf, jnp.float32)
        s_ref[...] = jnp.zeros((1, n), jnp.float32)

    s = jax.lax.dot_general(
        l_ref[...], f_ref[...], (((1,), (1,)), ((), ())),
        preferred_element_type=jnp.float32)
    out_ref[...] = s

    ones = jnp.ones((1, c_block), jnp.float32)

    def _update(sm):
        bmax = jnp.max(sm, axis=0, keepdims=True)
        m_old = m_ref[...]
        m_new = jnp.maximum(m_old, bmax)
        m_ref[...] = m_new
        e = jnp.exp(sm - m_new)
        esum = jax.lax.dot_general(
            ones, e, (((1,), (0,)), ((), ())),
            preferred_element_type=jnp.float32)
        s_ref[...] = s_ref[...] * jnp.exp(m_old - m_new) + esum

    @pl.when(i < nblk - 1)
    def _main():
        _update(s)

    @pl.when(i == nblk - 1)
    def _edge():
        row = jax.lax.broadcasted_iota(jnp.int32, (c_block, 1), 0) + i * c_block
        _update(jnp.where(row < num_classes, s, -jnp.inf))
        lse = m_ref[...] + jnp.log(s_ref[...])
        lsesum_ref[...] = jnp.sum(lse).reshape(1, 1)


def _loss_body(f_ref, g_ref, a_ref, loss_ref, *, n):
    tsum = jnp.sum(f_ref[...] * g_ref[...])
    loss_ref[...] = ((a_ref[0, 0] - tsum) / n).reshape(1, 1)


def kernel(features, targets, lut):
    n, nf = features.shape
    num_classes = lut.shape[0]
    c_block = _C_BLOCK
    nblk = pl.cdiv(num_classes, c_block)

    fs = features * jnp.float32(_SCALAR)
    g_rows = _make_target_row_gather(n, nf)(targets.astype(jnp.int32), lut)

    scaled_t, lse_sum = pl.pallas_call(
        functools.partial(_oim_body, num_classes=num_classes, c_block=c_block),
        grid=(nblk,),
        in_specs=[
            pl.BlockSpec((n, nf), lambda i: (0, 0)),
            pl.BlockSpec((c_block, nf), lambda i: (i, 0)),
        ],
        out_specs=[
            pl.BlockSpec((c_block, n), lambda i: (i, 0)),
            pl.BlockSpec((1, 1), lambda i: (0, 0)),
        ],
        out_shape=[
            jax.ShapeDtypeStruct((num_classes, n), jnp.float32),
            jax.ShapeDtypeStruct((1, 1), jnp.float32),
        ],
        scratch_shapes=[
            pltpu.VMEM((1, n), jnp.float32),
            pltpu.VMEM((1, n), jnp.float32),
        ],
        compiler_params=pltpu.CompilerParams(
            vmem_limit_bytes=60 * 1024 * 1024),
    )(fs, lut)

    loss = pl.pallas_call(
        functools.partial(_loss_body, n=n),
        out_shape=jax.ShapeDtypeStruct((1, 1), jnp.float32),
    )(fs, g_rows, lse_sum)
    return (loss.reshape(()), scaled_t.T)
